# baseline (speedup 1.0000x reference)
import functools

import jax
import jax.numpy as jnp
from jax import lax
from jax.experimental import pallas as pl
from jax.experimental.pallas import tpu as pltpu
from jax.experimental.pallas import tpu_sc as plsc

_N = 10000
_DEG = 32
_D = 128
_L = 16
_NG = _D // _L
_NC = 2
_NS = 16
_NW = _NC * _NS
_NPW = 320
_PN = _NW * _NPW
_C = 2
_EC = _C * _DEG
_CHUNKS = _NPW // _C
_NBUF = 2

_mesh = plsc.VectorSubcoreMesh(core_axis_name="c", subcore_axis_name="s")


@functools.partial(
    pl.kernel,
    out_type=jax.ShapeDtypeStruct((_PN, _D), jnp.float32),
    mesh=_mesh,
    scratch_types=(
        [pltpu.VMEM_SHARED((_PN, _D), jnp.float32)]
        + [pltpu.VMEM((_CHUNKS, _EC), jnp.int32)]
        + [pltpu.VMEM((_EC, _D), jnp.float32) for _ in range(_NBUF)]
        + [pltpu.VMEM((_C, _D), jnp.float32) for _ in range(_NBUF)]
        + [pltpu.SemaphoreType.DMA for _ in range(2 * _NBUF)]
    ),
)
def _aggregate(x_hbm, idx_hbm, out_hbm, x_sp, idx_all, r0, r1,
               a0, a1, g0, g1, o0, o1):
    sid = lax.axis_index("s")
    wid = sid * _NC + lax.axis_index("c")
    node0 = wid * _NPW
    rows = (r0, r1)
    accs = (a0, a1)
    gsems = (g0, g1)
    osems = (o0, o1)

    per_sub = _PN // 64 // _NS
    piece0 = sid * per_sub

    for b in range(2):
        pltpu.async_copy(x_hbm.at[pl.ds((piece0 + b) * 64, 64)],
                         rows[b], gsems[b])

    def stage_piece(j0, carry):
        for b in range(2):
            j = j0 * 2 + b
            off = (piece0 + j) * 64
            pltpu.make_async_copy(x_hbm.at[pl.ds(off, 64)],
                                  rows[b], gsems[b]).wait()
            pltpu.sync_copy(rows[b], x_sp.at[pl.ds(off, 64)])

            @pl.when(j + 2 < per_sub)
            def _next():
                pltpu.async_copy(x_hbm.at[pl.ds((piece0 + j + 2) * 64, 64)],
                                 rows[b], gsems[b])
        return carry

    lax.fori_loop(0, per_sub // 2, stage_piece, 0)

    pltpu.sync_copy(idx_hbm.at[pl.ds(wid * _CHUNKS, _CHUNKS)], idx_all)

    plsc.subcore_barrier()

    for b in range(_NBUF):
        pltpu.async_copy(x_sp.at[idx_all.at[b]], rows[b], gsems[b])

    def accumulate(rb, ab):
        for n in range(_C):
            for g in range(_NG):
                sl = pl.ds(g * _L, _L)
                acc = [rb[n * _DEG + j, sl] for j in range(4)]
                for r in range(4, _DEG, 4):
                    for j in range(4):
                        acc[j] = acc[j] + rb[n * _DEG + r + j, sl]
                ab[n, sl] = (acc[0] + acc[1]) + (acc[2] + acc[3])

    def step(k0, carry):
        for b in range(_NBUF):
            k = k0 * _NBUF + b
            rb, ab, gs, os = rows[b], accs[b], gsems[b], osems[b]
            base = node0 + k * _C
            pltpu.make_async_copy(x_sp.at[idx_all.at[k]], rb, gs).wait()

            @pl.when(k0 > 0)
            def _wait_prev_out():
                pltpu.make_async_copy(ab, out_hbm.at[pl.ds(base, _C)], os).wait()

            accumulate(rb, ab)
            pltpu.async_copy(ab, out_hbm.at[pl.ds(base, _C)], os)

            @pl.when(k + _NBUF < _CHUNKS)
            def _prefetch():
                pltpu.async_copy(x_sp.at[idx_all.at[k + _NBUF]], rows[b], gsems[b])
        return carry

    lax.fori_loop(0, _CHUNKS // _NBUF, step, 0)

    for b in range(_NBUF):
        pltpu.make_async_copy(accs[b], out_hbm.at[pl.ds(node0, _C)], osems[b]).wait()


def _mm_body(x_ref, w_ref, o_ref):
    o_ref[...] = jnp.dot(x_ref[...], w_ref[...], preferred_element_type=jnp.float32)


_BM = 1024


def _matmul(xp, w):
    return pl.pallas_call(
        _mm_body,
        grid=(_PN // _BM,),
        in_specs=[
            pl.BlockSpec((_BM, _D), lambda i: (i, 0)),
            pl.BlockSpec((_D, _D), lambda i: (0, 0)),
        ],
        out_specs=pl.BlockSpec((_BM, _D), lambda i: (i, 0)),
        out_shape=jax.ShapeDtypeStruct((_PN, _D), jnp.float32),
    )(xp, w)


def kernel(X, weights, row_pointers, column_index, blockPartition, edgeToColumn,
           edgeToRow, hybrid_type, row_nzr, col_nzr, output):
    e = column_index.shape[0]
    idx_pad = jnp.concatenate(
        [column_index, jnp.zeros((_PN * _DEG - e,), jnp.int32)])
    idx2d = idx_pad.reshape(_PN * _DEG // _EC, _EC)
    x_pad = jnp.concatenate(
        [X, jnp.zeros((_PN - _N, _D), jnp.float32)])
    xp = _aggregate(x_pad, idx2d)
    y = _matmul(xp, weights)
    return y[:_N]


# speedup vs baseline: 1.2034x; 1.2034x over previous
import functools

import jax
import jax.numpy as jnp
from jax import lax
from jax.experimental import pallas as pl
from jax.experimental.pallas import tpu as pltpu
from jax.experimental.pallas import tpu_sc as plsc

_N = 10000
_DEG = 32
_D = 128
_L = 16
_NG = _D // _L
_NC = 2
_NS = 16
_NW = _NC * _NS
_NPW = 320
_PN = _NW * _NPW
_C = 2
_EC = _C * _DEG
_CHUNKS = _NPW // _C
_NBUF = 2

_mesh = plsc.VectorSubcoreMesh(core_axis_name="c", subcore_axis_name="s")


@functools.partial(
    pl.kernel,
    out_type=jax.ShapeDtypeStruct((_PN, _D), jnp.float32),
    mesh=_mesh,
    scratch_types=(
        [pltpu.VMEM_SHARED((_PN, _D), jnp.float32)]
        + [pltpu.VMEM((_CHUNKS, _EC), jnp.int32)]
        + [pltpu.VMEM((_EC, _D), jnp.float32) for _ in range(_NBUF)]
        + [pltpu.VMEM((_C, _D), jnp.float32) for _ in range(_NBUF)]
        + [pltpu.SemaphoreType.DMA for _ in range(2 * _NBUF)]
    ),
)
def _aggregate(x_hbm, idx_hbm, out_hbm, x_sp, idx_all, r0, r1,
               a0, a1, g0, g1, o0, o1):
    sid = lax.axis_index("s")
    wid = sid * _NC + lax.axis_index("c")
    node0 = wid * _NPW
    rows = (r0, r1)
    accs = (a0, a1)
    gsems = (g0, g1)
    osems = (o0, o1)

    per_sub = _PN // 64 // _NS
    piece0 = sid * per_sub

    for b in range(2):
        pltpu.async_copy(x_hbm.at[pl.ds((piece0 + b) * 64, 64)],
                         rows[b], gsems[b])

    def stage_piece(j0, carry):
        for b in range(2):
            j = j0 * 2 + b
            off = (piece0 + j) * 64
            pltpu.make_async_copy(x_hbm.at[pl.ds(off, 64)],
                                  rows[b], gsems[b]).wait()
            pltpu.sync_copy(rows[b], x_sp.at[pl.ds(off, 64)])

            @pl.when(j + 2 < per_sub)
            def _next():
                pltpu.async_copy(x_hbm.at[pl.ds((piece0 + j + 2) * 64, 64)],
                                 rows[b], gsems[b])
        return carry

    lax.fori_loop(0, per_sub // 2, stage_piece, 0)

    pltpu.sync_copy(idx_hbm.at[pl.ds(wid * _CHUNKS, _CHUNKS)], idx_all)

    plsc.subcore_barrier()

    for b in range(_NBUF):
        pltpu.async_copy(x_sp.at[idx_all.at[b]], rows[b], gsems[b])

    def accumulate(rb, ab):
        for n in range(_C):
            for g in range(_NG):
                sl = pl.ds(g * _L, _L)
                acc = rb[n * _DEG, sl]
                for r in range(1, _DEG):
                    acc = acc + rb[n * _DEG + r, sl]
                ab[n, sl] = acc

    def step(k0, carry):
        for b in range(_NBUF):
            k = k0 * _NBUF + b
            rb, ab, gs, os = rows[b], accs[b], gsems[b], osems[b]
            base = node0 + k * _C
            pltpu.make_async_copy(x_sp.at[idx_all.at[k]], rb, gs).wait()

            @pl.when(k0 > 0)
            def _wait_prev_out():
                pltpu.make_async_copy(ab, out_hbm.at[pl.ds(base, _C)], os).wait()

            accumulate(rb, ab)
            pltpu.async_copy(ab, out_hbm.at[pl.ds(base, _C)], os)

            @pl.when(k + _NBUF < _CHUNKS)
            def _prefetch():
                pltpu.async_copy(x_sp.at[idx_all.at[k + _NBUF]], rows[b], gsems[b])
        return carry

    lax.fori_loop(0, _CHUNKS // _NBUF, step, 0)

    for b in range(_NBUF):
        pltpu.make_async_copy(accs[b], out_hbm.at[pl.ds(node0, _C)], osems[b]).wait()


def _mm_body(x_ref, w_ref, o_ref):
    o_ref[...] = jnp.dot(x_ref[...], w_ref[...], preferred_element_type=jnp.float32)


_BM = 1024


def _matmul(xp, w):
    return pl.pallas_call(
        _mm_body,
        grid=(_PN // _BM,),
        in_specs=[
            pl.BlockSpec((_BM, _D), lambda i: (i, 0)),
            pl.BlockSpec((_D, _D), lambda i: (0, 0)),
        ],
        out_specs=pl.BlockSpec((_BM, _D), lambda i: (i, 0)),
        out_shape=jax.ShapeDtypeStruct((_PN, _D), jnp.float32),
    )(xp, w)


def kernel(X, weights, row_pointers, column_index, blockPartition, edgeToColumn,
           edgeToRow, hybrid_type, row_nzr, col_nzr, output):
    e = column_index.shape[0]
    idx_pad = jnp.concatenate(
        [column_index, jnp.zeros((_PN * _DEG - e,), jnp.int32)])
    idx2d = idx_pad.reshape(_PN * _DEG // _EC, _EC)
    x_pad = jnp.concatenate(
        [X, jnp.zeros((_PN - _N, _D), jnp.float32)])
    xp = _aggregate(x_pad, idx2d)
    y = _matmul(xp, weights)
    return y[:_N]


# speedup vs baseline: 1.8015x; 1.4969x over previous
import functools

import jax
import jax.numpy as jnp
from jax import lax
from jax.experimental import pallas as pl
from jax.experimental.pallas import tpu as pltpu
from jax.experimental.pallas import tpu_sc as plsc

_N = 10000
_DEG = 32
_D = 128
_L = 16
_NG = _D // _L
_NC = 2
_NS = 16
_NW = _NC * _NS
_NPW = 320
_PN = _NW * _NPW
_C = 4
_EC = _C * _DEG
_CHUNKS = _NPW // _C
_NBUF = 2

_mesh = plsc.VectorSubcoreMesh(core_axis_name="c", subcore_axis_name="s")


@functools.partial(
    pl.kernel,
    out_type=jax.ShapeDtypeStruct((_PN, _D), jnp.float32),
    mesh=_mesh,
    scratch_types=(
        [pltpu.VMEM_SHARED((_PN, _D), jnp.float32)]
        + [pltpu.VMEM((_CHUNKS, _EC), jnp.int32)]
        + [pltpu.VMEM((_EC, _D), jnp.float32) for _ in range(_NBUF)]
        + [pltpu.VMEM((_C, _D), jnp.float32) for _ in range(_NBUF)]
        + [pltpu.SemaphoreType.DMA for _ in range(2 * _NBUF)]
    ),
)
def _aggregate(x_hbm, idx_hbm, out_hbm, x_sp, idx_all, r0, r1,
               a0, a1, g0, g1, o0, o1):
    sid = lax.axis_index("s")
    wid = sid * _NC + lax.axis_index("c")
    node0 = wid * _NPW
    rows = (r0, r1)
    accs = (a0, a1)
    gsems = (g0, g1)
    osems = (o0, o1)

    per_sub = _PN // _EC // _NS
    piece0 = sid * per_sub

    for b in range(min(2, per_sub)):
        pltpu.async_copy(x_hbm.at[pl.ds((piece0 + b) * _EC, _EC)],
                         rows[b], gsems[b])

    for j in range(per_sub):
        b = j % 2
        off = (piece0 + j) * _EC
        pltpu.make_async_copy(x_hbm.at[pl.ds(off, _EC)],
                              rows[b], gsems[b]).wait()
        pltpu.sync_copy(rows[b], x_sp.at[pl.ds(off, _EC)])
        if j + 2 < per_sub:
            pltpu.async_copy(x_hbm.at[pl.ds((piece0 + j + 2) * _EC, _EC)],
                             rows[b], gsems[b])

    pltpu.sync_copy(idx_hbm.at[pl.ds(wid * _CHUNKS, _CHUNKS)], idx_all)

    plsc.subcore_barrier()

    for b in range(_NBUF):
        pltpu.async_copy(x_sp.at[idx_all.at[b]], rows[b], gsems[b])

    def accumulate(rb, ab):
        def node_body(n, carry):
            for g in range(_NG):
                sl = pl.ds(g * _L, _L)
                acc = rb[n * _DEG, sl]
                for r in range(1, _DEG):
                    acc = acc + rb[n * _DEG + r, sl]
                ab[n, sl] = acc
            return carry
        lax.fori_loop(0, _C, node_body, 0)

    def step(k0, carry):
        for b in range(_NBUF):
            k = k0 * _NBUF + b
            rb, ab, gs, os = rows[b], accs[b], gsems[b], osems[b]
            base = node0 + k * _C
            pltpu.make_async_copy(x_sp.at[idx_all.at[k]], rb, gs).wait()

            @pl.when(k0 > 0)
            def _wait_prev_out():
                pltpu.make_async_copy(ab, out_hbm.at[pl.ds(base, _C)], os).wait()

            accumulate(rb, ab)
            pltpu.async_copy(ab, out_hbm.at[pl.ds(base, _C)], os)

            @pl.when(k + _NBUF < _CHUNKS)
            def _prefetch():
                pltpu.async_copy(x_sp.at[idx_all.at[k + _NBUF]], rows[b], gsems[b])
        return carry

    lax.fori_loop(0, _CHUNKS // _NBUF, step, 0)

    for b in range(_NBUF):
        pltpu.make_async_copy(accs[b], out_hbm.at[pl.ds(node0, _C)], osems[b]).wait()


def _mm_body(x_ref, w_ref, o_ref):
    o_ref[...] = jnp.dot(x_ref[...], w_ref[...], preferred_element_type=jnp.float32)


_BM = 1024


def _matmul(xp, w):
    return pl.pallas_call(
        _mm_body,
        grid=(_PN // _BM,),
        in_specs=[
            pl.BlockSpec((_BM, _D), lambda i: (i, 0)),
            pl.BlockSpec((_D, _D), lambda i: (0, 0)),
        ],
        out_specs=pl.BlockSpec((_BM, _D), lambda i: (i, 0)),
        out_shape=jax.ShapeDtypeStruct((_PN, _D), jnp.float32),
    )(xp, w)


def kernel(X, weights, row_pointers, column_index, blockPartition, edgeToColumn,
           edgeToRow, hybrid_type, row_nzr, col_nzr, output):
    e = column_index.shape[0]
    idx_pad = jnp.concatenate(
        [column_index, jnp.zeros((_PN * _DEG - e,), jnp.int32)])
    idx2d = idx_pad.reshape(_PN * _DEG // _EC, _EC)
    x_pad = jnp.concatenate(
        [X, jnp.zeros((_PN - _N, _D), jnp.float32)])
    xp = _aggregate(x_pad, idx2d)
    y = _matmul(xp, weights)
    return y[:_N]
